# Initial kernel scaffold; baseline (speedup 1.0000x reference)
import jax
import jax.numpy as jnp
from jax.experimental import pallas as pl


def kernel(x, edge_index, edge_attr, batch, reactant_natoms, reactant_nbonds, reactant_mw, Wn1, bn1, Wn2, bn2, Wn3, bn3, We1, be1, We2, be2, We3, be3, Wnt, bnt, Wmt, bmt, Wf1, bf1, Wf2, bf2, Wf3, bf3, Wf4, bf4):
    raise NotImplementedError("write your pallas kernel here")


# speedup vs baseline: 5.5825x; 5.5825x over previous
import functools

import jax
import jax.numpy as jnp
from jax import lax
from jax.experimental import pallas as pl
from jax.experimental.pallas import tpu as pltpu
from jax.experimental.pallas import tpu_sc as plsc

N = 10000
E = 160000
EP = 163840
G = 64
NW = 32
NB = 40
BW = 128
ACC_R = 10240
NPAD = 10240

_f32 = jnp.float32


def _preload_indices(src_hbm, dst_hbm, sidx, didx, w):
    pltpu.sync_copy(src_hbm.at[pl.ds(w * NB, NB)], sidx)
    pltpu.sync_copy(dst_hbm.at[pl.ds(w * NB, NB)], didx)


def _zero_acc(acc, zbuf, s, zr=128):
    def zb(i, c):
        pltpu.sync_copy(zbuf, acc.at[pl.ds(s * 640 + i * zr, zr)])
        return c
    lax.fori_loop(0, 640 // zr, zb, 0)


def _flush_acc(acc, buf, out2d, s):
    def piece(off, rows):
        pltpu.sync_copy(acc.at[pl.ds(off, rows)], buf.at[pl.ds(0, rows)])
        pltpu.sync_copy(buf.at[pl.ds(0, rows)], out2d.at[pl.ds(off, rows)])

    @pl.when(s < 15)
    def _main():
        def fl(i, c):
            piece(s * 640 + i * 128, 128)
            return c
        lax.fori_loop(0, 5, fl, 0)

    @pl.when(s == 15)
    def _tail():
        def fl(i, c):
            piece(9600 + i * 128, 128)
            return c
        lax.fori_loop(0, 3, fl, 0)
        piece(9984, 16)


@functools.lru_cache(maxsize=None)
def _prop_kernel(K):
    mesh = plsc.VectorSubcoreMesh(core_axis_name="c", subcore_axis_name="s",
                                  num_cores=2, num_subcores=16)

    @functools.partial(
        pl.kernel,
        out_type=jax.ShapeDtypeStruct((2, K, N, 128), _f32),
        mesh=mesh,
        scratch_types=[
            pltpu.VMEM((NB, BW), jnp.int32),
            pltpu.VMEM((NB, BW), jnp.int32),
            pltpu.VMEM((BW, 128), _f32),
            pltpu.VMEM((BW, 128), _f32),
            pltpu.VMEM((32, 128), _f32),
            pltpu.VMEM_SHARED((ACC_R, 128), _f32),
            pltpu.SemaphoreType.DMA,
            pltpu.SemaphoreType.DMA,
        ],
    )
    def prop(u_hbm, src_hbm, dst_hbm, zer_hbm, out_hbm,
             sidx, didx, buf0, buf1, zbuf, acc, sem0, sem1):
        c = lax.axis_index("c")
        s = lax.axis_index("s")
        w = c * 16 + s
        _preload_indices(src_hbm, dst_hbm, sidx, didx, w)
        pltpu.sync_copy(zer_hbm, zbuf)
        for k in range(K):
            _zero_acc(acc, zbuf, s, 32)
            plsc.subcore_barrier()
            tbl = u_hbm.at[k]
            pltpu.make_async_copy(tbl.at[sidx.at[0]], buf0, sem0).start()

            def body(i, carry):
                j0 = 2 * i
                pltpu.make_async_copy(tbl.at[sidx.at[j0 + 1]], buf1, sem1).start()
                pltpu.make_async_copy(tbl.at[sidx.at[j0]], buf0, sem0).wait()
                pltpu.sync_copy(buf0, acc.at[didx.at[j0]], add=True)
                pltpu.make_async_copy(tbl.at[sidx.at[j0 + 2]], buf0, sem0).start()
                pltpu.make_async_copy(tbl.at[sidx.at[j0 + 1]], buf1, sem1).wait()
                pltpu.sync_copy(buf1, acc.at[didx.at[j0 + 1]], add=True)
                return carry

            lax.fori_loop(0, NB // 2 - 1, body, 0)
            pltpu.make_async_copy(tbl.at[sidx.at[NB - 1]], buf1, sem1).start()
            pltpu.make_async_copy(tbl.at[sidx.at[NB - 2]], buf0, sem0).wait()
            pltpu.sync_copy(buf0, acc.at[didx.at[NB - 2]], add=True)
            pltpu.make_async_copy(tbl.at[sidx.at[NB - 1]], buf1, sem1).wait()
            pltpu.sync_copy(buf1, acc.at[didx.at[NB - 1]], add=True)
            plsc.subcore_barrier()
            _flush_acc(acc, buf0, out_hbm.at[c].at[k], s)
            plsc.subcore_barrier()

    return prop


def _sc_prop(u, src_r, dst_r):
    return _prop_kernel(u.shape[0])(u, src_r, dst_r, jnp.zeros((32, 128), _f32))


@functools.lru_cache(maxsize=None)
def _hist_kernel():
    mesh = plsc.VectorSubcoreMesh(core_axis_name="c", subcore_axis_name="s",
                                  num_cores=2, num_subcores=16)

    @functools.partial(
        pl.kernel,
        out_type=jax.ShapeDtypeStruct((2, 2, N, 128), _f32),
        mesh=mesh,
        scratch_types=[
            pltpu.VMEM((NB, BW), jnp.int32),
            pltpu.VMEM((NB, BW), jnp.int32),
            pltpu.VMEM((BW, 128), _f32),
            pltpu.VMEM((BW, 128), _f32),
            pltpu.VMEM((32, 128), _f32),
            pltpu.VMEM_SHARED((ACC_R, 128), _f32),
        ],
    )
    def hist(src_hbm, dst_hbm, ones_hbm, zer_hbm, out_hbm,
             sidx, didx, ones, buf, zbuf, acc):
        c = lax.axis_index("c")
        s = lax.axis_index("s")
        w = c * 16 + s
        _preload_indices(src_hbm, dst_hbm, sidx, didx, w)
        pltpu.sync_copy(ones_hbm, ones)
        pltpu.sync_copy(zer_hbm, zbuf)
        for p, idx in ((0, didx), (1, sidx)):
            _zero_acc(acc, zbuf, s, 32)
            plsc.subcore_barrier()

            def body(j, cy):
                pltpu.sync_copy(ones, acc.at[idx.at[j]], add=True)
                return cy
            lax.fori_loop(0, NB, body, 0)
            plsc.subcore_barrier()
            _flush_acc(acc, buf, out_hbm.at[c].at[p], s)
            plsc.subcore_barrier()

    return hist


def _sc_hist(src_r, dst_r):
    return _hist_kernel()(src_r, dst_r, jnp.ones((BW, 128), _f32),
                          jnp.zeros((32, 128), _f32))


@functools.lru_cache(maxsize=None)
def _agg_kernel():
    mesh = plsc.VectorSubcoreMesh(core_axis_name="c", subcore_axis_name="s",
                                  num_cores=2, num_subcores=16)

    @functools.partial(
        pl.kernel,
        out_type=jax.ShapeDtypeStruct((2, N, 128), _f32),
        mesh=mesh,
        scratch_types=[
            pltpu.VMEM((NB, BW), jnp.int32),
            pltpu.VMEM((BW, 128), _f32),
            pltpu.VMEM((BW, 128), _f32),
            pltpu.VMEM((32, 128), _f32),
            pltpu.VMEM_SHARED((ACC_R, 128), _f32),
            pltpu.SemaphoreType.DMA,
            pltpu.SemaphoreType.DMA,
        ],
    )
    def agg(e_hbm, src_hbm, zer_hbm, out_hbm,
            sidx, buf0, buf1, zbuf, acc, sem0, sem1):
        c = lax.axis_index("c")
        s = lax.axis_index("s")
        w = c * 16 + s
        pltpu.sync_copy(src_hbm.at[pl.ds(w * NB, NB)], sidx)
        pltpu.sync_copy(zer_hbm, zbuf)
        _zero_acc(acc, zbuf, s, 32)
        plsc.subcore_barrier()
        pltpu.make_async_copy(e_hbm.at[pl.ds(w * NB * BW, BW)], buf0, sem0).start()

        def body(i, cy):
            j0 = 2 * i
            pltpu.make_async_copy(
                e_hbm.at[pl.ds((w * NB + j0 + 1) * BW, BW)], buf1, sem1).start()
            pltpu.make_async_copy(
                e_hbm.at[pl.ds((w * NB + j0) * BW, BW)], buf0, sem0).wait()
            pltpu.sync_copy(buf0, acc.at[sidx.at[j0]], add=True)
            pltpu.make_async_copy(
                e_hbm.at[pl.ds((w * NB + j0 + 2) * BW, BW)], buf0, sem0).start()
            pltpu.make_async_copy(
                e_hbm.at[pl.ds((w * NB + j0 + 1) * BW, BW)], buf1, sem1).wait()
            pltpu.sync_copy(buf1, acc.at[sidx.at[j0 + 1]], add=True)
            return cy

        lax.fori_loop(0, NB // 2 - 1, body, 0)
        pltpu.make_async_copy(
            e_hbm.at[pl.ds((w * NB + NB - 1) * BW, BW)], buf1, sem1).start()
        pltpu.make_async_copy(
            e_hbm.at[pl.ds((w * NB + NB - 2) * BW, BW)], buf0, sem0).wait()
        pltpu.sync_copy(buf0, acc.at[sidx.at[NB - 2]], add=True)
        pltpu.make_async_copy(
            e_hbm.at[pl.ds((w * NB + NB - 1) * BW, BW)], buf1, sem1).wait()
        pltpu.sync_copy(buf1, acc.at[sidx.at[NB - 1]], add=True)
        plsc.subcore_barrier()
        _flush_acc(acc, buf0, out_hbm.at[c], s)

    return agg


def _sc_agg(e3p, src_r):
    return _agg_kernel()(e3p, src_r, jnp.zeros((32, 128), _f32))


_NBLK = 400
_EBLK = 1024


def _deg_stats(histb):
    deg = histb[0, 0, :, 0:1] + histb[1, 0, :, 0:1] + 1.0
    return lax.rsqrt(deg), 1.0 / deg


def _full(shape):
    return pl.BlockSpec(shape, lambda i: tuple(0 for _ in shape))


def _tck_pre(x, e0n, hist):
    def body(x_ref, e0_ref, h_ref, u_ref):
        dis, _ = _deg_stats(h_ref[...])
        xb = x_ref[...]
        c2 = jnp.concatenate(
            [dis * e0_ref[...], jnp.zeros((_NBLK, 112), _f32)], axis=1)
        u_ref[...] = jnp.stack(
            [dis * xb[:, 0:128], dis * xb[:, 128:256], c2])

    return pl.pallas_call(
        body,
        grid=(N // _NBLK,),
        in_specs=[
            pl.BlockSpec((_NBLK, 256), lambda i: (i, 0)),
            pl.BlockSpec((_NBLK, 16), lambda i: (i, 0)),
            pl.BlockSpec((2, 2, _NBLK, 128), lambda i: (0, 0, i, 0)),
        ],
        out_specs=pl.BlockSpec((3, _NBLK, 128), lambda i: (0, i, 0)),
        out_shape=jax.ShapeDtypeStruct((3, N, 128), _f32),
    )(x, e0n, hist)


def _edge_mask_stats(histb, i):
    rows = i * _EBLK + lax.broadcasted_iota(jnp.int32, (_EBLK, 1), 0)
    mask = rows < N
    dis, invd = _deg_stats(histb)
    return mask, jnp.where(mask, dis, 1.0), jnp.where(mask, invd, 1.0)


def _clamp_node(i):
    return jnp.minimum(i, NPAD // _EBLK - 1)


def _tck_e1(e0p, p0e_p, hist_p, We1, be1):
    def body(e0_ref, p_ref, h_ref, w_ref, b_ref, out_ref):
        i = pl.program_id(0)
        mask, dis, invd = _edge_mask_stats(h_ref[...], i)
        s0e = jnp.where(mask, p_ref[0] + p_ref[1], 0.0)
        pre = dis * s0e + invd * e0_ref[...]
        out_ref[...] = jnp.tanh(
            jnp.dot(pre, w_ref[...], preferred_element_type=_f32) + b_ref[...])

    return pl.pallas_call(
        body,
        grid=(EP // _EBLK,),
        in_specs=[
            pl.BlockSpec((_EBLK, 16), lambda i: (i, 0)),
            pl.BlockSpec((2, _EBLK, 16), lambda i: (0, _clamp_node(i), 0)),
            pl.BlockSpec((2, 2, _EBLK, 128), lambda i: (0, 0, _clamp_node(i), 0)),
            _full((16, 64)),
            _full((1, 64)),
        ],
        out_specs=pl.BlockSpec((_EBLK, 64), lambda i: (i, 0)),
        out_shape=jax.ShapeDtypeStruct((EP, 64), _f32),
    )(e0p, p0e_p, hist_p, We1, be1)


def _tck_n1(p0, x, hist, e1p, Wn1, bn1):
    def body(p_ref, x_ref, h_ref, e1_ref, w_ref, b_ref, h1_ref, u_ref):
        dis, invd = _deg_stats(h_ref[...])
        s0n = jnp.concatenate(
            [p_ref[0, 0] + p_ref[1, 0], p_ref[0, 1] + p_ref[1, 1]], axis=1)
        z = dis * s0n + invd * x_ref[...]
        h1 = jnp.tanh(jnp.dot(z, w_ref[...], preferred_element_type=_f32)
                      + b_ref[...])
        h1_ref[...] = h1
        u = dis * h1
        ue = jnp.concatenate(
            [dis * e1_ref[...], jnp.zeros((_NBLK, 64), _f32)], axis=1)
        u_ref[...] = jnp.stack(
            [u[:, 0:128], u[:, 128:256], u[:, 256:384], u[:, 384:512], ue])

    return pl.pallas_call(
        body,
        grid=(N // _NBLK,),
        in_specs=[
            pl.BlockSpec((2, 3, _NBLK, 128), lambda i: (0, 0, i, 0)),
            pl.BlockSpec((_NBLK, 256), lambda i: (i, 0)),
            pl.BlockSpec((2, 2, _NBLK, 128), lambda i: (0, 0, i, 0)),
            pl.BlockSpec((_NBLK, 64), lambda i: (i, 0)),
            _full((256, 512)),
            _full((1, 512)),
        ],
        out_specs=[
            pl.BlockSpec((_NBLK, 512), lambda i: (i, 0)),
            pl.BlockSpec((5, _NBLK, 128), lambda i: (0, i, 0)),
        ],
        out_shape=[
            jax.ShapeDtypeStruct((N, 512), _f32),
            jax.ShapeDtypeStruct((5, N, 128), _f32),
        ],
    )(p0, x, hist, e1p, Wn1, bn1)


def _tck_e2(e1p, p1e_p, hist_p, We2, be2, We3):
    def body(e1_ref, p_ref, h_ref, w2_ref, b2_ref, w3_ref, out_ref):
        i = pl.program_id(0)
        mask, dis, invd = _edge_mask_stats(h_ref[...], i)
        s1e = jnp.where(mask, p_ref[0] + p_ref[1], 0.0)
        pre = dis * s1e + invd * e1_ref[...]
        e2 = jnp.tanh(jnp.dot(pre, w2_ref[...], preferred_element_type=_f32)
                      + b2_ref[...])
        out_ref[...] = jnp.dot(e2, w3_ref[...], preferred_element_type=_f32)

    return pl.pallas_call(
        body,
        grid=(EP // _EBLK,),
        in_specs=[
            pl.BlockSpec((_EBLK, 64), lambda i: (i, 0)),
            pl.BlockSpec((2, _EBLK, 64), lambda i: (0, _clamp_node(i), 0)),
            pl.BlockSpec((2, 2, _EBLK, 128), lambda i: (0, 0, _clamp_node(i), 0)),
            _full((64, 64)),
            _full((1, 64)),
            _full((64, 16)),
        ],
        out_specs=pl.BlockSpec((_EBLK, 16), lambda i: (i, 0)),
        out_shape=jax.ShapeDtypeStruct((EP, 16), _f32),
    )(e1p, p1e_p, hist_p, We2, be2, We3)


def _tck_n2(p1, h1, hist, gep, Wn2, bn2, Wn3):
    def body(p_ref, h1_ref, h_ref, ge_ref, w2_ref, b2_ref, w3_ref,
             gn_ref, u_ref):
        dis, invd = _deg_stats(h_ref[...])
        s1n = jnp.concatenate(
            [p_ref[0, 0] + p_ref[1, 0], p_ref[0, 1] + p_ref[1, 1],
             p_ref[0, 2] + p_ref[1, 2], p_ref[0, 3] + p_ref[1, 3]], axis=1)
        z = dis * s1n + invd * h1_ref[...]
        h2 = jnp.tanh(jnp.dot(z, w2_ref[...], preferred_element_type=_f32)
                      + b2_ref[...])
        g = jnp.dot(h2, w3_ref[...], preferred_element_type=_f32)
        gn_ref[...] = g
        u = dis * g
        ue = jnp.concatenate(
            [dis * ge_ref[...], jnp.zeros((_NBLK, 112), _f32)], axis=1)
        u_ref[...] = jnp.stack([u[:, 0:128], u[:, 128:256], ue])

    return pl.pallas_call(
        body,
        grid=(N // _NBLK,),
        in_specs=[
            pl.BlockSpec((2, 4, _NBLK, 128), lambda i: (0, 0, i, 0)),
            pl.BlockSpec((_NBLK, 512), lambda i: (i, 0)),
            pl.BlockSpec((2, 2, _NBLK, 128), lambda i: (0, 0, i, 0)),
            pl.BlockSpec((_NBLK, 16), lambda i: (i, 0)),
            _full((512, 512)),
            _full((1, 512)),
            _full((512, 256)),
        ],
        out_specs=[
            pl.BlockSpec((_NBLK, 256), lambda i: (i, 0)),
            pl.BlockSpec((3, _NBLK, 128), lambda i: (0, i, 0)),
        ],
        out_shape=[
            jax.ShapeDtypeStruct((N, 256), _f32),
            jax.ShapeDtypeStruct((3, N, 128), _f32),
        ],
    )(p1, h1, hist, gep, Wn2, bn2, Wn3)


def _tck_e3(gep, p2e_p, hist_p, be3):
    def body(ge_ref, p_ref, h_ref, b_ref, out_ref):
        i = pl.program_id(0)
        mask, dis, invd = _edge_mask_stats(h_ref[...], i)
        s2e = jnp.where(mask, p_ref[0] + p_ref[1], 0.0)
        e3 = dis * s2e + invd * ge_ref[...] + b_ref[...]
        out_ref[...] = jnp.concatenate(
            [e3, jnp.zeros((_EBLK, 112), _f32)], axis=1)

    return pl.pallas_call(
        body,
        grid=(EP // _EBLK,),
        in_specs=[
            pl.BlockSpec((_EBLK, 16), lambda i: (i, 0)),
            pl.BlockSpec((2, _EBLK, 16), lambda i: (0, _clamp_node(i), 0)),
            pl.BlockSpec((2, 2, _EBLK, 128), lambda i: (0, 0, _clamp_node(i), 0)),
            _full((1, 16)),
        ],
        out_specs=pl.BlockSpec((_EBLK, 128), lambda i: (i, 0)),
        out_shape=jax.ShapeDtypeStruct((EP, 128), _f32),
    )(gep, p2e_p, hist_p, be3)


def _tck_final(p2, gn, hist, aggp, batch2, bn3, Wnt, bnt, Wmt_x, Wmt_a, bmt):
    def body(p_ref, gn_ref, h_ref, a_ref, b_ref, bn3_ref, wnt_ref, bnt_ref,
             wmx_ref, wma_ref, bmt_ref, psum_ref, pcnt_ref):
        i = pl.program_id(0)
        histb = h_ref[...]
        dis, invd = _deg_stats(histb)
        s2n = jnp.concatenate(
            [p_ref[0, 0] + p_ref[1, 0], p_ref[0, 1] + p_ref[1, 1]], axis=1)
        h3 = dis * s2n + invd * gn_ref[...] + bn3_ref[...]
        xt = jnp.dot(h3, wnt_ref[...], preferred_element_type=_f32) + bnt_ref[...]
        cnt_src = histb[0, 1, :, 0:1] + histb[1, 1, :, 0:1]
        agg = (a_ref[0, :, 0:16] + a_ref[1, :, 0:16]) / jnp.maximum(cnt_src, 1.0)
        msg = (jnp.dot(xt, wmx_ref[...], preferred_element_type=_f32)
               + jnp.dot(agg, wma_ref[...], preferred_element_type=_f32)
               + bmt_ref[...])
        hf = h3 + msg
        onehot = (b_ref[...] == lax.broadcasted_iota(
            jnp.int32, (_NBLK, G), 1)).astype(_f32)

        @pl.when(i == 0)
        def _init():
            psum_ref[...] = jnp.zeros_like(psum_ref)
            pcnt_ref[...] = jnp.zeros_like(pcnt_ref)

        psum_ref[...] += lax.dot_general(
            onehot, hf, (((0,), (0,)), ((), ())), preferred_element_type=_f32)
        pcnt_ref[...] += jnp.sum(onehot, axis=0)[:, None]

    return pl.pallas_call(
        body,
        grid=(N // _NBLK,),
        in_specs=[
            pl.BlockSpec((2, 2, _NBLK, 128), lambda i: (0, 0, i, 0)),
            pl.BlockSpec((_NBLK, 256), lambda i: (i, 0)),
            pl.BlockSpec((2, 2, _NBLK, 128), lambda i: (0, 0, i, 0)),
            pl.BlockSpec((2, _NBLK, 128), lambda i: (0, i, 0)),
            pl.BlockSpec((_NBLK, 1), lambda i: (i, 0)),
            _full((1, 256)),
            _full((256, 256)),
            _full((1, 256)),
            _full((256, 256)),
            _full((16, 256)),
            _full((1, 256)),
        ],
        out_specs=[
            pl.BlockSpec((G, 256), lambda i: (0, 0)),
            pl.BlockSpec((G, 1), lambda i: (0, 0)),
        ],
        out_shape=[
            jax.ShapeDtypeStruct((G, 256), _f32),
            jax.ShapeDtypeStruct((G, 1), _f32),
        ],
    )(p2, gn, hist, aggp, batch2, bn3, Wnt, bnt, Wmt_x, Wmt_a, bmt)


def _tck_mlp(psum, pcnt, na, nb, mw, Wf1a, Wf1b, bf1, Wf2, bf2, Wf3, bf3,
             Wf4, bf4):
    def body(ps_ref, pc_ref, na_ref, nb_ref, mw_ref, w1a_ref, w1b_ref, b1_ref,
             w2_ref, b2_ref, w3_ref, b3_ref, w4_ref, b4_ref, out_ref):
        cnt = pc_ref[...]
        pooled = ps_ref[...] / jnp.maximum(cnt, 1.0)
        xg = (na_ref[...] + nb_ref[...] + mw_ref[...]) * (1.0 / 3.0)
        xg = jnp.where(cnt > 0.0, xg, 0.0)
        o = jax.nn.relu(
            jnp.dot(pooled, w1a_ref[...], preferred_element_type=_f32)
            + jnp.dot(xg, w1b_ref[...], preferred_element_type=_f32)
            + b1_ref[...])
        o = jax.nn.relu(jnp.dot(o, w2_ref[...], preferred_element_type=_f32)
                        + b2_ref[...])
        o = jax.nn.relu(jnp.dot(o, w3_ref[...], preferred_element_type=_f32)
                        + b3_ref[...])
        out_ref[...] = (jnp.dot(o, w4_ref[...], preferred_element_type=_f32)
                        + b4_ref[...])

    return pl.pallas_call(
        body,
        out_shape=jax.ShapeDtypeStruct((G, 1), _f32),
    )(psum, pcnt, na, nb, mw, Wf1a, Wf1b, bf1, Wf2, bf2, Wf3, bf3, Wf4, bf4)


def _pad_nodes(a, axis):
    pads = [(0, 0)] * a.ndim
    pads[axis] = (0, NPAD - N)
    return jnp.pad(a, pads)


def kernel(x, edge_index, edge_attr, batch, reactant_natoms, reactant_nbonds,
           reactant_mw, Wn1, bn1, Wn2, bn2, Wn3, bn3, We1, be1, We2, be2,
           We3, be3, Wnt, bnt, Wmt, bmt, Wf1, bf1, Wf2, bf2, Wf3, bf3,
           Wf4, bf4):
    src = edge_index[0]
    dst = edge_index[1]
    padi = jnp.full((EP - E,), N, jnp.int32)
    src_g = jnp.concatenate([src, jnp.zeros((EP - E,), jnp.int32)]).reshape(NW * NB, BW)
    src_s = jnp.concatenate([src, padi]).reshape(NW * NB, BW)
    dst_s = jnp.concatenate([dst, padi]).reshape(NW * NB, BW)

    hist = _sc_hist(src_s, dst_s)
    hist_p = _pad_nodes(hist, 2)
    e0p = jnp.concatenate(
        [edge_attr, jnp.zeros((EP - E, 16), _f32)], axis=0)

    u0 = _tck_pre(x, edge_attr[:N], hist)
    p0 = _sc_prop(u0, src_g, dst_s)

    p0e_p = _pad_nodes(p0[:, 2, :, :16], 1)
    e1p = _tck_e1(e0p, p0e_p, hist_p, We1, be1.reshape(1, 64))
    h1, u1 = _tck_n1(p0, x, hist, e1p[:N], Wn1, bn1.reshape(1, 512))
    p1 = _sc_prop(u1, src_g, dst_s)

    p1e_p = _pad_nodes(p1[:, 4, :, :64], 1)
    gep = _tck_e2(e1p, p1e_p, hist_p, We2, be2.reshape(1, 64), We3)
    gn, u2 = _tck_n2(p1, h1, hist, gep[:N], Wn2, bn2.reshape(1, 512), Wn3)
    p2 = _sc_prop(u2, src_g, dst_s)

    p2e_p = _pad_nodes(p2[:, 2, :, :16], 1)
    e3p = _tck_e3(gep, p2e_p, hist_p, be3.reshape(1, 16))
    aggp = _sc_agg(e3p, src_s)

    psum, pcnt = _tck_final(
        p2, gn, hist, aggp, batch.reshape(N, 1), bn3.reshape(1, 256),
        Wnt, bnt.reshape(1, 256), Wmt[:256], Wmt[256:], bmt.reshape(1, 256))
    return _tck_mlp(
        psum, pcnt, reactant_natoms.reshape(G, 1), reactant_nbonds.reshape(G, 1),
        reactant_mw.reshape(G, 1), Wf1[:256], Wf1[256:], bf1.reshape(1, 128),
        Wf2, bf2.reshape(1, 64), Wf3, bf3.reshape(1, 32), Wf4, bf4.reshape(1, 1))
